# baseline (speedup 1.0000x reference)
import jax
import jax.numpy as jnp
from jax import lax
from jax.experimental import pallas as pl
from jax.experimental.pallas import tpu as pltpu
from jax.experimental.pallas import tpu_sc as plsc

N = 10000
E = 320000
D = 128
H = 256
FH = 128
NC, NS = 2, 16
NROW = 10240
ROWS_PT = NROW // NS
KJ = 16
CHUNK = KJ * 128
EPT = 20480
EPAD = NS * EPT
PAD = EPAD - E
NCHUNK = EPT // CHUNK
IRPT = EPT // 128
IRSLOT = EPAD // 128

_MESH = plsc.VectorSubcoreMesh(
    core_axis_name="c", subcore_axis_name="s", num_cores=NC, num_subcores=NS
)


def _sc_scatter_body(table, srcoff, dst, out, srcbuf, dstbuf, rows, acc,
                     semg, sems, semi):
    c = lax.axis_index("c")
    s = lax.axis_index("s")
    pltpu.sync_copy(
        table.at[pl.ds(c * NROW + s * ROWS_PT, ROWS_PT)],
        acc.at[pl.ds(s * ROWS_PT, ROWS_PT)],
    )
    plsc.subcore_barrier()

    irbase = c * IRSLOT + s * IRPT

    pltpu.sync_copy(srcoff.at[pl.ds(irbase, KJ)], srcbuf.at[0])
    pltpu.sync_copy(dst.at[pl.ds(irbase, KJ)], dstbuf.at[0])

    def chunk(i, carry):
        p = lax.rem(i, 2)
        rbn = irbase + jnp.minimum(i + 1, NCHUNK - 1) * KJ
        pf1 = pltpu.async_copy(srcoff.at[pl.ds(rbn, KJ)], srcbuf.at[1 - p], semi)
        pf2 = pltpu.async_copy(dst.at[pl.ds(rbn, KJ)], dstbuf.at[1 - p], semi)
        gd = [None] * KJ
        sd = [None] * KJ
        for j in range(KJ):
            b = j % 2
            if j >= 2:
                sd[j - 2].wait()
            gd[j] = pltpu.async_copy(
                table.at[srcbuf.at[p, j]], rows.at[b], semg
            )
            if j >= 1:
                gd[j - 1].wait()
                sd[j - 1] = pltpu.async_copy(
                    rows.at[1 - b], acc.at[dstbuf.at[p, j - 1]], sems, add=True
                )
        gd[KJ - 1].wait()
        sd[KJ - 1] = pltpu.async_copy(
            rows.at[(KJ - 1) % 2], acc.at[dstbuf.at[p, KJ - 1]], sems, add=True
        )
        sd[KJ - 2].wait()
        sd[KJ - 1].wait()
        pf1.wait()
        pf2.wait()
        return carry

    lax.fori_loop(0, NCHUNK, chunk, 0)

    plsc.subcore_barrier()
    pltpu.sync_copy(
        acc.at[pl.ds(s * ROWS_PT, ROWS_PT)],
        out.at[pl.ds(c * NROW + s * ROWS_PT, ROWS_PT)],
    )


_sc_scatter = pl.kernel(
    _sc_scatter_body,
    out_type=jax.ShapeDtypeStruct((2 * NROW, FH), jnp.float32),
    mesh=_MESH,
    scratch_types=[
        pltpu.VMEM((2, KJ, 128), jnp.int32),
        pltpu.VMEM((2, KJ, 128), jnp.int32),
        pltpu.VMEM((2, 128, FH), jnp.float32),
        pltpu.VMEM_SHARED((NROW, FH), jnp.float32),
        pltpu.SemaphoreType.DMA,
        pltpu.SemaphoreType.DMA,
        pltpu.SemaphoreType.DMA,
    ],
)


def _sc_deg_body(dstcat, zeros_h, ones_h, out, idxbuf, onesbuf, acc, sem,
                 semi):
    c = lax.axis_index("c")
    s = lax.axis_index("s")
    pltpu.sync_copy(
        zeros_h.at[pl.ds(s * ROWS_PT, ROWS_PT)],
        acc.at[pl.ds(s * ROWS_PT, ROWS_PT)],
    )
    pltpu.sync_copy(ones_h, onesbuf)
    plsc.subcore_barrier()

    irbase = c * IRSLOT + s * IRPT
    pltpu.sync_copy(dstcat.at[pl.ds(irbase, KJ)], idxbuf.at[0])

    def chunk(i, carry):
        p = lax.rem(i, 2)
        rbn = irbase + jnp.minimum(i + 1, NCHUNK - 1) * KJ
        pf = pltpu.async_copy(dstcat.at[pl.ds(rbn, KJ)], idxbuf.at[1 - p], semi)
        cps = [
            pltpu.async_copy(onesbuf, acc.at[idxbuf.at[p, j]], sem, add=True)
            for j in range(KJ)
        ]
        for cp in cps:
            cp.wait()
        pf.wait()
        return carry

    lax.fori_loop(0, NCHUNK, chunk, 0)

    plsc.subcore_barrier()
    pltpu.sync_copy(
        acc.at[pl.ds(s * ROWS_PT, ROWS_PT)],
        out.at[pl.ds(c * NROW + s * ROWS_PT, ROWS_PT)],
    )


_sc_deg = pl.kernel(
    _sc_deg_body,
    out_type=jax.ShapeDtypeStruct((2 * NROW, FH), jnp.float32),
    mesh=_MESH,
    scratch_types=[
        pltpu.VMEM((2, KJ, 128), jnp.int32),
        pltpu.VMEM((128, FH), jnp.float32),
        pltpu.VMEM_SHARED((NROW, FH), jnp.float32),
        pltpu.SemaphoreType.DMA,
        pltpu.SemaphoreType.DMA,
    ],
)


RB = 512
NRB = NROW // RB


def _mm_first(x, W, deg):
    def body(x_ref, w_ref, deg_ref, o_ref):
        dinv = lax.rsqrt(deg_ref[...])
        h = jnp.dot(x_ref[...], w_ref[...], preferred_element_type=jnp.float32)
        o_ref[0] = h * dinv

    return pl.pallas_call(
        body,
        grid=(2, NRB),
        in_specs=[
            pl.BlockSpec((RB, D), lambda c, r: (r, 0)),
            pl.BlockSpec((D, FH), lambda c, r: (0, c)),
            pl.BlockSpec((RB, 1), lambda c, r: (r, 0)),
        ],
        out_specs=pl.BlockSpec((1, RB, FH), lambda c, r: (c, r, 0)),
        out_shape=jax.ShapeDtypeStruct((2, NROW, FH), jnp.float32),
    )(x, W, deg)


def _mm_mid(S, deg, b, W, n_slots):
    fin = 2 * S.shape[2]
    fow = W.shape[1] // n_slots

    def body(s_ref, deg_ref, b_ref, w_ref, o_ref):
        dinv = lax.rsqrt(deg_ref[...])
        xb = jnp.concatenate([s_ref[0], s_ref[1]], axis=-1) * dinv + b_ref[...]
        h = jnp.dot(xb, w_ref[...], preferred_element_type=jnp.float32)
        o_ref[0] = h * dinv

    return pl.pallas_call(
        body,
        grid=(n_slots, NRB),
        in_specs=[
            pl.BlockSpec((2, RB, S.shape[2]), lambda c, r: (0, r, 0)),
            pl.BlockSpec((RB, 1), lambda c, r: (r, 0)),
            pl.BlockSpec((1, fin), lambda c, r: (0, 0)),
            pl.BlockSpec((fin, fow), lambda c, r: (0, c)),
        ],
        out_specs=pl.BlockSpec((1, RB, fow), lambda c, r: (c, r, 0)),
        out_shape=jax.ShapeDtypeStruct((n_slots, NROW, fow), jnp.float32),
    )(S, deg, b, W)


def _final(S, deg, b):
    def body(s_ref, deg_ref, b_ref, o_ref):
        o_ref[...] = s_ref[...] * lax.rsqrt(deg_ref[...]) + b_ref[...]

    return pl.pallas_call(
        body,
        grid=(NRB,),
        in_specs=[
            pl.BlockSpec((RB, D), lambda r: (r, 0)),
            pl.BlockSpec((RB, 1), lambda r: (r, 0)),
            pl.BlockSpec((1, D), lambda r: (0, 0)),
        ],
        out_specs=pl.BlockSpec((RB, D), lambda r: (r, 0)),
        out_shape=jax.ShapeDtypeStruct((NROW, D), jnp.float32),
    )(S, deg, b)


def kernel(x, aug_x, edge_index1, edge_index2,
           W1, b1, W2, b2, W3, b3, W4, b4, W5, b5, W6, b6):
    ar = jnp.arange(PAD, dtype=jnp.int32)
    srcpad = ar % N
    dstpad = N + (ar % (NROW - N))

    def pad_edges(ei):
        return (jnp.concatenate([ei[0], srcpad]),
                jnp.concatenate([ei[1], dstpad]))

    s1p, d1p = pad_edges(edge_index1)
    s2p, d2p = pad_edges(edge_index2)

    def r2(a):
        return a.reshape(-1, 128)

    srcoff_g1 = r2(jnp.concatenate([s1p, s1p + NROW]))
    dstd_g1 = r2(jnp.concatenate([d1p, d1p]))
    srcoff_g2 = r2(jnp.concatenate([s2p, s2p + NROW]))
    dstd_g2 = r2(jnp.concatenate([d2p, d2p]))
    srcoff_3 = r2(jnp.concatenate([s1p, s2p + NROW]))
    dst_cat = r2(jnp.concatenate([d1p, d2p]))

    zeros_h = jnp.zeros((NROW, FH), jnp.float32)
    ones_h = jnp.ones((128, FH), jnp.float32)
    deg_out = _sc_deg(dst_cat, zeros_h, ones_h)
    deg1 = deg_out[0:N, 0:1] + 1.0
    deg2 = deg_out[NROW:NROW + N, 0:1] + 1.0

    b1r, b2r, b3r = b1.reshape(1, -1), b2.reshape(1, -1), b3.reshape(1, -1)
    b4r, b5r, b6r = b4.reshape(1, -1), b5.reshape(1, -1), b6.reshape(1, -1)

    g1 = _mm_first(x, W1, deg1)
    S1 = _sc_scatter(g1.reshape(2 * NROW, FH), srcoff_g1, dstd_g1)
    g2 = _mm_mid(S1.reshape(2, NROW, FH), deg1, b1r, W2, 2)
    S2 = _sc_scatter(g2.reshape(2 * NROW, FH), srcoff_g1, dstd_g1)
    g3 = _mm_mid(S2.reshape(2, NROW, FH), deg1, b2r, W3, 1)

    g4 = _mm_first(aug_x, W4, deg2)
    S4 = _sc_scatter(g4.reshape(2 * NROW, FH), srcoff_g2, dstd_g2)
    g5 = _mm_mid(S4.reshape(2, NROW, FH), deg2, b4r, W5, 2)
    S5 = _sc_scatter(g5.reshape(2 * NROW, FH), srcoff_g2, dstd_g2)
    g6 = _mm_mid(S5.reshape(2, NROW, FH), deg2, b5r, W6, 1)

    t3 = jnp.concatenate([g3[0], g6[0]], axis=0)
    S3 = _sc_scatter(t3, srcoff_3, dst_cat)
    x1 = _final(S3[0:NROW], deg1, b3r)[0:N]
    x2 = _final(S3[NROW:2 * NROW], deg2, b6r)[0:N]
    return (x1, x2)


# speedup vs baseline: 1.0950x; 1.0950x over previous
import jax
import jax.numpy as jnp
from jax import lax
from jax.experimental import pallas as pl
from jax.experimental.pallas import tpu as pltpu
from jax.experimental.pallas import tpu_sc as plsc

N = 10000
E = 320000
D = 128
H = 256
FH = 128
NC, NS = 2, 16
NROW = 10240
ROWS_PT = NROW // NS
KJ = 16
CHUNK = KJ * 128
EPT = 20480
EPAD = NS * EPT
PAD = EPAD - E
NCHUNK = EPT // CHUNK
IRPT = EPT // 128
IRSLOT = EPAD // 128

_MESH = plsc.VectorSubcoreMesh(
    core_axis_name="c", subcore_axis_name="s", num_cores=NC, num_subcores=NS
)


def _sc_scatter_body(table, srcoff, dst, out, srcbuf, dstbuf, rows, acc,
                     semg, sems, semi):
    c = lax.axis_index("c")
    s = lax.axis_index("s")
    pltpu.sync_copy(
        table.at[pl.ds(c * NROW + s * ROWS_PT, ROWS_PT)],
        acc.at[pl.ds(s * ROWS_PT, ROWS_PT)],
    )
    plsc.subcore_barrier()

    irbase = c * IRSLOT + s * IRPT

    pltpu.sync_copy(srcoff.at[pl.ds(irbase, KJ)], srcbuf.at[0])
    pltpu.sync_copy(dst.at[pl.ds(irbase, KJ)], dstbuf.at[0])

    def chunk(i, carry):
        p = lax.rem(i, 2)
        rbn = irbase + jnp.minimum(i + 1, NCHUNK - 1) * KJ
        pf1 = pltpu.async_copy(srcoff.at[pl.ds(rbn, KJ)], srcbuf.at[1 - p], semi)
        pf2 = pltpu.async_copy(dst.at[pl.ds(rbn, KJ)], dstbuf.at[1 - p], semi)
        gd = [None] * KJ
        sd = [None] * KJ
        for j in range(KJ):
            b = j % 2
            if j >= 2:
                sd[j - 2].wait()
            gd[j] = pltpu.async_copy(
                table.at[srcbuf.at[p, j]], rows.at[b], semg
            )
            if j >= 1:
                gd[j - 1].wait()
                sd[j - 1] = pltpu.async_copy(
                    rows.at[1 - b], acc.at[dstbuf.at[p, j - 1]], sems, add=True
                )
        gd[KJ - 1].wait()
        sd[KJ - 1] = pltpu.async_copy(
            rows.at[(KJ - 1) % 2], acc.at[dstbuf.at[p, KJ - 1]], sems, add=True
        )
        sd[KJ - 2].wait()
        sd[KJ - 1].wait()
        pf1.wait()
        pf2.wait()
        return carry

    lax.fori_loop(0, NCHUNK, chunk, 0)

    plsc.subcore_barrier()
    pltpu.sync_copy(
        acc.at[pl.ds(s * ROWS_PT, ROWS_PT)],
        out.at[pl.ds(c * NROW + s * ROWS_PT, ROWS_PT)],
    )


_sc_scatter = pl.kernel(
    _sc_scatter_body,
    out_type=jax.ShapeDtypeStruct((2 * NROW, FH), jnp.float32),
    mesh=_MESH,
    scratch_types=[
        pltpu.VMEM((2, KJ, 128), jnp.int32),
        pltpu.VMEM((2, KJ, 128), jnp.int32),
        pltpu.VMEM((2, 128, FH), jnp.float32),
        pltpu.VMEM_SHARED((NROW, FH), jnp.float32),
        pltpu.SemaphoreType.DMA,
        pltpu.SemaphoreType.DMA,
        pltpu.SemaphoreType.DMA,
    ],
)


def _sc_deg_body(dstcat, zeros_h, ones_h, out, idxbuf, onesbuf, acc, sem,
                 semi):
    c = lax.axis_index("c")
    s = lax.axis_index("s")
    pltpu.sync_copy(
        zeros_h.at[pl.ds(s * ROWS_PT, ROWS_PT)],
        acc.at[pl.ds(s * ROWS_PT, ROWS_PT)],
    )
    pltpu.sync_copy(ones_h, onesbuf)
    plsc.subcore_barrier()

    irbase = c * IRSLOT + s * IRPT
    pltpu.sync_copy(dstcat.at[pl.ds(irbase, KJ)], idxbuf.at[0])

    def chunk(i, carry):
        p = lax.rem(i, 2)
        rbn = irbase + jnp.minimum(i + 1, NCHUNK - 1) * KJ
        pf = pltpu.async_copy(dstcat.at[pl.ds(rbn, KJ)], idxbuf.at[1 - p], semi)
        cps = [
            pltpu.async_copy(onesbuf, acc.at[idxbuf.at[p, j]], sem, add=True)
            for j in range(KJ)
        ]
        for cp in cps:
            cp.wait()
        pf.wait()
        return carry

    lax.fori_loop(0, NCHUNK, chunk, 0)

    plsc.subcore_barrier()
    pltpu.sync_copy(
        acc.at[pl.ds(s * ROWS_PT, ROWS_PT)],
        out.at[pl.ds(c * NROW + s * ROWS_PT, ROWS_PT)],
    )


_sc_deg = pl.kernel(
    _sc_deg_body,
    out_type=jax.ShapeDtypeStruct((2 * NROW,), jnp.float32),
    mesh=_MESH,
    scratch_types=[
        pltpu.VMEM((2, KJ, 128), jnp.int32),
        pltpu.VMEM((128,), jnp.float32),
        pltpu.VMEM_SHARED((NROW,), jnp.float32),
        pltpu.SemaphoreType.DMA,
        pltpu.SemaphoreType.DMA,
    ],
)


RB = 512
NRB = NROW // RB


def _mm_first(x, W, deg):
    def body(x_ref, w_ref, deg_ref, o_ref):
        dinv = lax.rsqrt(deg_ref[...])
        h = jnp.dot(x_ref[...], w_ref[...], preferred_element_type=jnp.float32)
        o_ref[0] = h * dinv

    return pl.pallas_call(
        body,
        grid=(2, NRB),
        in_specs=[
            pl.BlockSpec((RB, D), lambda c, r: (r, 0)),
            pl.BlockSpec((D, FH), lambda c, r: (0, c)),
            pl.BlockSpec((RB, 1), lambda c, r: (r, 0)),
        ],
        out_specs=pl.BlockSpec((1, RB, FH), lambda c, r: (c, r, 0)),
        out_shape=jax.ShapeDtypeStruct((2, NROW, FH), jnp.float32),
    )(x, W, deg)


def _mm_mid(S, deg, b, W, n_slots):
    fin = 2 * S.shape[2]
    fow = W.shape[1] // n_slots

    def body(s_ref, deg_ref, b_ref, w_ref, o_ref):
        dinv = lax.rsqrt(deg_ref[...])
        xb = jnp.concatenate([s_ref[0], s_ref[1]], axis=-1) * dinv + b_ref[...]
        h = jnp.dot(xb, w_ref[...], preferred_element_type=jnp.float32)
        o_ref[0] = h * dinv

    return pl.pallas_call(
        body,
        grid=(n_slots, NRB),
        in_specs=[
            pl.BlockSpec((2, RB, S.shape[2]), lambda c, r: (0, r, 0)),
            pl.BlockSpec((RB, 1), lambda c, r: (r, 0)),
            pl.BlockSpec((1, fin), lambda c, r: (0, 0)),
            pl.BlockSpec((fin, fow), lambda c, r: (0, c)),
        ],
        out_specs=pl.BlockSpec((1, RB, fow), lambda c, r: (c, r, 0)),
        out_shape=jax.ShapeDtypeStruct((n_slots, NROW, fow), jnp.float32),
    )(S, deg, b, W)


def _final(S, deg, b):
    def body(s_ref, deg_ref, b_ref, o_ref):
        o_ref[...] = s_ref[...] * lax.rsqrt(deg_ref[...]) + b_ref[...]

    return pl.pallas_call(
        body,
        grid=(NRB,),
        in_specs=[
            pl.BlockSpec((RB, D), lambda r: (r, 0)),
            pl.BlockSpec((RB, 1), lambda r: (r, 0)),
            pl.BlockSpec((1, D), lambda r: (0, 0)),
        ],
        out_specs=pl.BlockSpec((RB, D), lambda r: (r, 0)),
        out_shape=jax.ShapeDtypeStruct((NROW, D), jnp.float32),
    )(S, deg, b)


def kernel(x, aug_x, edge_index1, edge_index2,
           W1, b1, W2, b2, W3, b3, W4, b4, W5, b5, W6, b6):
    ar = jnp.arange(PAD, dtype=jnp.int32)
    srcpad = ar % N
    dstpad = N + (ar % (NROW - N))

    def pad_edges(ei):
        return (jnp.concatenate([ei[0], srcpad]),
                jnp.concatenate([ei[1], dstpad]))

    s1p, d1p = pad_edges(edge_index1)
    s2p, d2p = pad_edges(edge_index2)

    def r2(a):
        return a.reshape(-1, 128)

    srcoff_g1 = r2(jnp.concatenate([s1p, s1p + NROW]))
    dstd_g1 = r2(jnp.concatenate([d1p, d1p]))
    srcoff_g2 = r2(jnp.concatenate([s2p, s2p + NROW]))
    dstd_g2 = r2(jnp.concatenate([d2p, d2p]))
    srcoff_3 = r2(jnp.concatenate([s1p, s2p + NROW]))
    dst_cat = r2(jnp.concatenate([d1p, d2p]))

    zeros_h = jnp.zeros((NROW,), jnp.float32)
    ones_h = jnp.ones((128,), jnp.float32)
    deg_out = _sc_deg(dst_cat, zeros_h, ones_h)
    deg1 = deg_out[0:N].reshape(N, 1) + 1.0
    deg2 = deg_out[NROW:NROW + N].reshape(N, 1) + 1.0

    b1r, b2r, b3r = b1.reshape(1, -1), b2.reshape(1, -1), b3.reshape(1, -1)
    b4r, b5r, b6r = b4.reshape(1, -1), b5.reshape(1, -1), b6.reshape(1, -1)

    g1 = _mm_first(x, W1, deg1)
    S1 = _sc_scatter(g1.reshape(2 * NROW, FH), srcoff_g1, dstd_g1)
    g2 = _mm_mid(S1.reshape(2, NROW, FH), deg1, b1r, W2, 2)
    S2 = _sc_scatter(g2.reshape(2 * NROW, FH), srcoff_g1, dstd_g1)
    g3 = _mm_mid(S2.reshape(2, NROW, FH), deg1, b2r, W3, 1)

    g4 = _mm_first(aug_x, W4, deg2)
    S4 = _sc_scatter(g4.reshape(2 * NROW, FH), srcoff_g2, dstd_g2)
    g5 = _mm_mid(S4.reshape(2, NROW, FH), deg2, b4r, W5, 2)
    S5 = _sc_scatter(g5.reshape(2 * NROW, FH), srcoff_g2, dstd_g2)
    g6 = _mm_mid(S5.reshape(2, NROW, FH), deg2, b5r, W6, 1)

    t3 = jnp.concatenate([g3[0], g6[0]], axis=0)
    S3 = _sc_scatter(t3, srcoff_3, dst_cat)
    x1 = _final(S3[0:NROW], deg1, b3r)[0:N]
    x2 = _final(S3[NROW:2 * NROW], deg2, b6r)[0:N]
    return (x1, x2)
